# baseline (speedup 1.0000x reference)
import functools

import jax
import jax.numpy as jnp
from jax import lax
from jax.experimental import pallas as pl
from jax.experimental.pallas import tpu as pltpu
from jax.experimental.pallas import tpu_sc as plsc

IN_F = 65536
OUT_F = 16384
NCON = 64
NC = 2
NS = 16
NW = NC * NS
ROWS_PER_W = OUT_F // NW
ROWS_PER_C = OUT_F // NC
GROUPS = ROWS_PER_W // 16
XSH = IN_F // NS


def _body(x_hbm, conn_hbm, op_hbm, out_hbm, x_sh, x_v, conn_v, op_v,
          out_v, sem_x, sem_conn, sem_op):
    cid = lax.axis_index("c")
    sid = lax.axis_index("s")
    base = (cid * NS + sid) * ROWS_PER_W

    xstage = pltpu.make_async_copy(
        x_hbm.at[pl.ds(sid * XSH, XSH)], x_sh.at[pl.ds(sid * XSH, XSH)],
        sem_x)
    conncopy = pltpu.make_async_copy(
        conn_hbm.at[pl.ds(base * NCON, ROWS_PER_W * NCON)], conn_v, sem_conn)
    opcopy = pltpu.make_async_copy(
        op_hbm.at[pl.ds(base, ROWS_PER_W)], op_v, sem_op)
    xstage.start()
    conncopy.start()
    opcopy.start()
    xstage.wait()
    plsc.subcore_barrier()

    xcopy = pltpu.make_async_copy(x_sh, x_v, sem_x)
    xcopy.start()
    xcopy.wait()
    conncopy.wait()
    opcopy.wait()

    lane = lax.iota(jnp.int32, 16)
    row_off = lane * NCON

    def group(g, carry):
        pos0 = (g * 16) * NCON + row_off

        def quad(q, accs):
            amins, amaxs = accs
            amins = list(amins)
            amaxs = list(amaxs)
            for a in range(4):
                ci = plsc.load_gather(conn_v, [pos0 + (q * 4 + a)])
                v = plsc.load_gather(x_v, [ci])
                amins[a] = jnp.minimum(amins[a], v)
                amaxs[a] = jnp.maximum(amaxs[a], v)
            return tuple(amins), tuple(amaxs)

        inf = jnp.full((16,), jnp.inf, jnp.float32)
        amins, amaxs = lax.fori_loop(
            0, NCON // 4, quad, ((inf,) * 4, (-inf,) * 4), unroll=2)
        mins = jnp.minimum(jnp.minimum(amins[0], amins[1]),
                           jnp.minimum(amins[2], amins[3]))
        maxs = jnp.maximum(jnp.maximum(amaxs[0], amaxs[1]),
                           jnp.maximum(amaxs[2], amaxs[3]))
        opv = op_v[pl.ds(g * 16, 16)]
        out_v[pl.ds(g * 16, 16)] = jnp.where(opv == 0, mins, maxs)
        return carry

    lax.fori_loop(0, GROUPS, group, 0)
    pltpu.sync_copy(out_v, out_hbm.at[pl.ds(base, ROWS_PER_W)])


@jax.jit
def kernel(x, connection_indices, operator_indices):
    conn = connection_indices.reshape(-1).astype(jnp.int32)
    op = operator_indices.astype(jnp.int32)

    mesh = plsc.VectorSubcoreMesh(core_axis_name="c", subcore_axis_name="s")
    call = functools.partial(
        pl.kernel,
        mesh=mesh,
        out_type=jax.ShapeDtypeStruct((OUT_F,), jnp.float32),
        compiler_params=pltpu.CompilerParams(needs_layout_passes=False),
        scratch_types=[
            pltpu.VMEM_SHARED((IN_F,), jnp.float32),
            pltpu.VMEM((IN_F,), jnp.float32),
            pltpu.VMEM((ROWS_PER_W * NCON,), jnp.int32),
            pltpu.VMEM((ROWS_PER_W,), jnp.int32),
            pltpu.VMEM((ROWS_PER_W,), jnp.float32),
            pltpu.SemaphoreType.DMA,
            pltpu.SemaphoreType.DMA,
            pltpu.SemaphoreType.DMA,
        ],
    )(_body)
    return call(x, conn, op)


# speedup vs baseline: 1.0087x; 1.0087x over previous
import functools

import jax
import jax.numpy as jnp
from jax import lax
from jax.experimental import pallas as pl
from jax.experimental.pallas import tpu as pltpu
from jax.experimental.pallas import tpu_sc as plsc

IN_F = 65536
OUT_F = 16384
NCON = 64
NC = 2
NS = 16
NW = NC * NS
ROWS_PER_W = OUT_F // NW
ROWS_PER_C = OUT_F // NC
GROUPS = ROWS_PER_W // 16
XSH = IN_F // NS


def _body(x_hbm, conn_hbm, op_hbm, out_hbm, x_sh, x_v, conn_v, op_v,
          out_v, sem_x, sem_conn, sem_op):
    cid = lax.axis_index("c")
    sid = lax.axis_index("s")
    base = (cid * NS + sid) * ROWS_PER_W

    xstage = pltpu.make_async_copy(
        x_hbm.at[pl.ds(sid * XSH, XSH)], x_sh.at[pl.ds(sid * XSH, XSH)],
        sem_x)
    conncopy = pltpu.make_async_copy(
        conn_hbm.at[pl.ds(base * NCON, ROWS_PER_W * NCON)], conn_v, sem_conn)
    opcopy = pltpu.make_async_copy(
        op_hbm.at[pl.ds(base, ROWS_PER_W)], op_v, sem_op)
    xstage.start()
    conncopy.start()
    opcopy.start()
    xstage.wait()
    plsc.subcore_barrier()

    xcopy = pltpu.make_async_copy(x_sh, x_v, sem_x)
    xcopy.start()
    xcopy.wait()
    conncopy.wait()
    opcopy.wait()

    lane = lax.iota(jnp.int32, 16)
    row_off = lane * NCON

    def group(g, carry):
        pos0 = (g * 16) * NCON + row_off

        def quad(q, accs):
            amins, amaxs = accs
            amins = list(amins)
            amaxs = list(amaxs)
            for a in range(4):
                ci = plsc.load_gather(conn_v, [pos0 + (q * 4 + a)])
                v = plsc.load_gather(x_v, [ci])
                amins[a] = jnp.minimum(amins[a], v)
                amaxs[a] = jnp.maximum(amaxs[a], v)
            return tuple(amins), tuple(amaxs)

        inf = jnp.full((16,), jnp.inf, jnp.float32)
        amins, amaxs = lax.fori_loop(
            0, NCON // 4, quad, ((inf,) * 4, (-inf,) * 4), unroll=4)
        mins = jnp.minimum(jnp.minimum(amins[0], amins[1]),
                           jnp.minimum(amins[2], amins[3]))
        maxs = jnp.maximum(jnp.maximum(amaxs[0], amaxs[1]),
                           jnp.maximum(amaxs[2], amaxs[3]))
        opv = op_v[pl.ds(g * 16, 16)]
        out_v[pl.ds(g * 16, 16)] = jnp.where(opv == 0, mins, maxs)
        return carry

    lax.fori_loop(0, GROUPS, group, 0)
    pltpu.sync_copy(out_v, out_hbm.at[pl.ds(base, ROWS_PER_W)])


@jax.jit
def kernel(x, connection_indices, operator_indices):
    conn = connection_indices.reshape(-1).astype(jnp.int32)
    op = operator_indices.astype(jnp.int32)

    mesh = plsc.VectorSubcoreMesh(core_axis_name="c", subcore_axis_name="s")
    call = functools.partial(
        pl.kernel,
        mesh=mesh,
        out_type=jax.ShapeDtypeStruct((OUT_F,), jnp.float32),
        compiler_params=pltpu.CompilerParams(needs_layout_passes=False),
        scratch_types=[
            pltpu.VMEM_SHARED((IN_F,), jnp.float32),
            pltpu.VMEM((IN_F,), jnp.float32),
            pltpu.VMEM((ROWS_PER_W * NCON,), jnp.int32),
            pltpu.VMEM((ROWS_PER_W,), jnp.int32),
            pltpu.VMEM((ROWS_PER_W,), jnp.float32),
            pltpu.SemaphoreType.DMA,
            pltpu.SemaphoreType.DMA,
            pltpu.SemaphoreType.DMA,
        ],
    )(_body)
    return call(x, conn, op)
